# baseline (speedup 1.0000x reference)
import functools

import jax
import jax.numpy as jnp
from jax import lax
from jax.experimental import pallas as pl
from jax.experimental.pallas import tpu as pltpu
from jax.experimental.pallas import tpu_sc as plsc

_NC = 2
_NS = 16
_NW = _NC * _NS
_LANES = 16

_NEG = -1e30


def _sc_pool(enc_pad, a0, a1, a2, emb2, cat0p, cat1p, cat2p, w_flat,
             B, L, LP, D):
    BPW = B // _NW
    DG = D // _LANES
    W2 = 2 * D
    CH0 = 104
    CH1 = L - CH0
    NV = LP // _LANES
    TAIL = L - (NV - 1) * _LANES
    NFULL = L // _LANES

    mesh = plsc.VectorSubcoreMesh(core_axis_name="c", subcore_axis_name="s")

    @functools.partial(
        pl.kernel,
        out_type=jax.ShapeDtypeStruct((B, D), jnp.float32),
        mesh=mesh,
        compiler_params=pltpu.CompilerParams(use_tc_tiling_on_sc=True,
                                             needs_layout_passes=False),
        scratch_types=[
            pltpu.VMEM((BPW * LP,), jnp.int32),
            pltpu.VMEM((LP,), jnp.int32),
            pltpu.VMEM((LP,), jnp.int32),
            pltpu.VMEM((L, W2), jnp.float32),
            pltpu.VMEM((L, W2), jnp.float32),
            pltpu.VMEM((BPW,), jnp.int32),
            pltpu.VMEM((BPW, W2), jnp.float32),
            pltpu.VMEM((BPW, D), jnp.float32),
            pltpu.VMEM((D,), jnp.float32),
            pltpu.VMEM((LP,), jnp.float32),
            pltpu.SemaphoreType.DMA,
            pltpu.SemaphoreType.DMA,
            pltpu.SemaphoreType.DMA,
        ],
    )
    def body(enc_hbm, a0_hbm, a1_hbm, a2_hbm, tab_hbm, c0_hbm, c1_hbm,
             c2_hbm, w_hbm, out_hbm, idx_v, dmx_a, dmx_b, rows_a, rows_b,
             cat_idx, cat_v, out_v, w_v, scores_v, sem_a, sem_b, sem_c):
        cid = lax.axis_index("c")
        sid = lax.axis_index("s")
        wid = sid * _NC + cid
        base = wid * BPW

        pltpu.sync_copy(enc_hbm.at[pl.ds(base * LP, BPW * LP)], idx_v)
        pltpu.sync_copy(w_hbm, w_v)

        lane = lax.iota(jnp.int32, _LANES)

        def cat_accum(first):
            def acc_body(r, _):
                for g in range(DG):
                    v = cat_v[r, pl.ds(g * 16, 16)]
                    if first:
                        out_v[r, pl.ds(g * 16, 16)] = v
                    else:
                        out_v[r, pl.ds(g * 16, 16)] = (
                            out_v[r, pl.ds(g * 16, 16)] + v)
                return 0
            lax.fori_loop(0, BPW, acc_body, 0, unroll=False)

        pltpu.sync_copy(a0_hbm.at[pl.ds(base, BPW)], cat_idx)
        pltpu.async_copy(c0_hbm.at[cat_idx], cat_v, sem_c).wait()
        cat_accum(True)
        pltpu.sync_copy(a1_hbm.at[pl.ds(base, BPW)], cat_idx)
        pltpu.async_copy(c1_hbm.at[cat_idx], cat_v, sem_c).wait()
        cat_accum(False)
        pltpu.sync_copy(a2_hbm.at[pl.ds(base, BPW)], cat_idx)
        pltpu.async_copy(c2_hbm.at[cat_idx], cat_v, sem_c).wait()
        cat_accum(False)

        w0 = w_v[pl.ds(0, _LANES)]
        w1 = w_v[pl.ds(16, _LANES)]
        w2 = w_v[pl.ds(32, _LANES)]
        w3 = w_v[pl.ds(48, _LANES)]

        def fill_dmx(r, dmx):
            ib = r * LP
            def fb(i, _):
                dmx[pl.ds(i * _LANES, _LANES)] = (
                    idx_v[pl.ds(ib + i * _LANES, _LANES)] >> 1)
                return 0
            lax.fori_loop(0, NV, fb, 0, unroll=False)

        def start_gather(buf, dmx, sem):
            pltpu.async_copy(tab_hbm.at[dmx.at[pl.ds(0, CH0)]],
                             buf.at[pl.ds(0, CH0)], sem)
            pltpu.async_copy(tab_hbm.at[dmx.at[pl.ds(CH0, CH1)]],
                             buf.at[pl.ds(CH0, CH1)], sem)

        def wait_gather(buf, dmx, sem):
            pltpu.make_async_copy(tab_hbm.at[dmx.at[pl.ds(0, CH0)]],
                                  buf.at[pl.ds(0, CH0)], sem).wait()
            pltpu.make_async_copy(tab_hbm.at[dmx.at[pl.ds(CH0, CH1)]],
                                  buf.at[pl.ds(CH0, CH1)], sem).wait()

        def rotate(v, sh):
            return jnp.take_along_axis(v, (lane + sh) & (_LANES - 1), axis=0)

        def allreduce_sum(v):
            for sh in (8, 4, 2, 1):
                v = v + rotate(v, sh)
            return v

        def allreduce_max(v):
            for sh in (8, 4, 2, 1):
                v = jnp.maximum(v, rotate(v, sh))
            return v

        def compute_row(buf, r):
            ibase = r * LP

            def tok_cols(poff16, u):
                pb = jnp.take_along_axis(
                    poff16, jnp.full((_LANES,), u, jnp.int32), axis=0)
                return [pb + (lane + g * 16) for g in range(DG)]

            def score_group(i, n_tok):
                poff16 = (idx_v[pl.ds(ibase + i * _LANES, _LANES)] & 1) << 6
                sv = jnp.full((_LANES,), _NEG, jnp.float32)
                for u in range(n_tok):
                    j = i * _LANES + u
                    row = jnp.full((_LANES,), j, jnp.int32)
                    c0, c1, c2, c3 = tok_cols(poff16, u)
                    v = (plsc.load_gather(buf, [row, c0]) * w0
                         + plsc.load_gather(buf, [row, c1]) * w1
                         + plsc.load_gather(buf, [row, c2]) * w2
                         + plsc.load_gather(buf, [row, c3]) * w3)
                    sv = jnp.where(lane == u, allreduce_sum(v), sv)
                scores_v[pl.ds(i * _LANES, _LANES)] = sv
                return sv

            def score_body(i, mv):
                return jnp.maximum(mv, score_group(i, _LANES))
            mv = lax.fori_loop(0, NFULL, score_body,
                               jnp.full((_LANES,), _NEG, jnp.float32),
                               unroll=False)
            mv = jnp.maximum(mv, score_group(NFULL, TAIL))
            m = allreduce_max(mv)

            def exp_body(i, sv):
                e = jnp.exp(scores_v[pl.ds(i * _LANES, _LANES)] - m)
                scores_v[pl.ds(i * _LANES, _LANES)] = e
                return sv + e
            sv = lax.fori_loop(0, NV, exp_body,
                               jnp.zeros((_LANES,), jnp.float32),
                               unroll=False)
            inv = 1.0 / allreduce_sum(sv)

            def wsum_group(i, n_tok, accs):
                poff16 = (idx_v[pl.ds(ibase + i * _LANES, _LANES)] & 1) << 6
                e16 = scores_v[pl.ds(i * _LANES, _LANES)]
                a0v, a1v, a2v, a3v = accs
                for u in range(n_tok):
                    j = i * _LANES + u
                    row = jnp.full((_LANES,), j, jnp.int32)
                    wb = jnp.take_along_axis(
                        e16, jnp.full((_LANES,), u, jnp.int32), axis=0)
                    c0, c1, c2, c3 = tok_cols(poff16, u)
                    a0v = a0v + wb * plsc.load_gather(buf, [row, c0])
                    a1v = a1v + wb * plsc.load_gather(buf, [row, c1])
                    a2v = a2v + wb * plsc.load_gather(buf, [row, c2])
                    a3v = a3v + wb * plsc.load_gather(buf, [row, c3])
                return (a0v, a1v, a2v, a3v)
            z16 = jnp.zeros((_LANES,), jnp.float32)
            accs = lax.fori_loop(0, NFULL,
                                 lambda i, accs: wsum_group(i, _LANES, accs),
                                 (z16, z16, z16, z16), unroll=False)
            accs = wsum_group(NFULL, TAIL, accs)

            for g in range(DG):
                out_v[r, pl.ds(g * 16, 16)] = (
                    out_v[r, pl.ds(g * 16, 16)] + accs[g] * inv)

        fill_dmx(0, dmx_a)
        start_gather(rows_a, dmx_a, sem_a)
        fill_dmx(1, dmx_b)
        start_gather(rows_b, dmx_b, sem_b)

        def row_pair(k, _):
            r = k * 2
            wait_gather(rows_a, dmx_a, sem_a)
            compute_row(rows_a, r)

            @pl.when(r + 2 < BPW)
            def _():
                fill_dmx(r + 2, dmx_a)
                start_gather(rows_a, dmx_a, sem_a)

            wait_gather(rows_b, dmx_b, sem_b)
            compute_row(rows_b, r + 1)

            @pl.when(r + 3 < BPW)
            def _():
                fill_dmx(r + 3, dmx_b)
                start_gather(rows_b, dmx_b, sem_b)
            return 0

        lax.fori_loop(0, BPW // 2, row_pair, 0, unroll=False)

        pltpu.sync_copy(out_v, out_hbm.at[pl.ds(base, BPW)])

    return body(enc_pad, a0, a1, a2, emb2, cat0p, cat1p, cat2p, w_flat)


def _tc_fc(x_in, fc_w, fc_b2):
    B, D = x_in.shape
    C = fc_w.shape[1]
    BLK = 512

    def body(x_ref, w_ref, b_ref, o_ref):
        o_ref[...] = (
            jnp.dot(x_ref[...], w_ref[...], preferred_element_type=jnp.float32)
            + b_ref[...])

    return pl.pallas_call(
        body,
        grid=(B // BLK,),
        in_specs=[
            pl.BlockSpec((BLK, D), lambda i: (i, 0)),
            pl.BlockSpec((D, C), lambda i: (0, 0)),
            pl.BlockSpec((1, C), lambda i: (0, 0)),
        ],
        out_specs=pl.BlockSpec((BLK, C), lambda i: (i, 0)),
        out_shape=jax.ShapeDtypeStruct((B, C), jnp.float32),
    )(x_in, fc_w, fc_b2)


def kernel(encoded_text, additional_inputs, emb_table, cat_table0, cat_table1,
           cat_table2, attn_w, fc_w, fc_b):
    B, L = encoded_text.shape
    V, D = emb_table.shape
    LP = ((L + 15) // 16) * 16
    enc = encoded_text.astype(jnp.int32)
    enc_pad = jnp.pad(enc, ((0, 0), (0, LP - L))).reshape(-1)
    a0 = additional_inputs[0].astype(jnp.int32)
    a1 = additional_inputs[1].astype(jnp.int32)
    a2 = additional_inputs[2].astype(jnp.int32)
    emb2 = emb_table.reshape(V // 2, 2 * D)
    pad = ((0, 0), (0, D))
    cat0p = jnp.pad(cat_table0, pad)
    cat1p = jnp.pad(cat_table1, pad)
    cat2p = jnp.pad(cat_table2, pad)
    w_flat = attn_w.reshape(-1)
    x_in = _sc_pool(enc_pad, a0, a1, a2, emb2, cat0p, cat1p, cat2p, w_flat,
                    B, L, LP, D)
    return _tc_fc(x_in, fc_w, fc_b.reshape(1, -1))


# speedup vs baseline: 3.3440x; 3.3440x over previous
import functools

import jax
import jax.numpy as jnp
from jax import lax
from jax.experimental import pallas as pl
from jax.experimental.pallas import tpu as pltpu
from jax.experimental.pallas import tpu_sc as plsc

_NC = 2
_NS = 16
_NW = _NC * _NS
_LANES = 16

_NEG = -1e30

_PB = 4096


def _repack_table(emb_table):
    V, D = emb_table.shape
    off_blk = V // (2 * _PB)
    off = _PB * off_blk
    nblk = (V - off + _PB - 1) // _PB
    rows = _PB * nblk

    def body(x1_ref, x2_ref, o_ref):
        t1 = jnp.transpose(x1_ref[...])
        t2 = jnp.transpose(x2_ref[...])
        o_ref[...] = jnp.concatenate([t1, t2], axis=1)

    embT = emb_table.T
    packed = pl.pallas_call(
        body,
        grid=(nblk,),
        in_specs=[
            pl.BlockSpec((D, _PB), lambda i: (0, i)),
            pl.BlockSpec((D, _PB), lambda i: (0, i + off_blk)),
        ],
        out_specs=pl.BlockSpec((_PB, 2 * D), lambda i: (i, 0)),
        out_shape=jax.ShapeDtypeStruct((rows, 2 * D), jnp.float32),
    )(embT, embT)
    return packed.reshape(2 * rows, D), off


def _sc_pool(enc_pad, a0, a1, a2, tab64, cat0, cat1, cat2, w_flat,
             B, L, LP, D, OFF):
    BPW = B // _NW
    DG = D // _LANES
    CH0 = 104
    CH1 = L - CH0
    NV = LP // _LANES
    TAIL = L - (NV - 1) * _LANES
    NFULL = L // _LANES

    mesh = plsc.VectorSubcoreMesh(core_axis_name="c", subcore_axis_name="s")

    @functools.partial(
        pl.kernel,
        out_type=jax.ShapeDtypeStruct((B, D), jnp.float32),
        mesh=mesh,
        compiler_params=pltpu.CompilerParams(use_tc_tiling_on_sc=False),
        scratch_types=[
            pltpu.VMEM((BPW * LP,), jnp.int32),
            pltpu.VMEM((LP,), jnp.int32),
            pltpu.VMEM((LP,), jnp.int32),
            pltpu.VMEM((L, D), jnp.float32),
            pltpu.VMEM((L, D), jnp.float32),
            pltpu.VMEM((BPW,), jnp.int32),
            pltpu.VMEM((BPW, D), jnp.float32),
            pltpu.VMEM((BPW, D), jnp.float32),
            pltpu.VMEM((BPW, D), jnp.float32),
            pltpu.VMEM((BPW, D), jnp.float32),
            pltpu.VMEM((D,), jnp.float32),
            pltpu.VMEM((LP,), jnp.float32),
            pltpu.SemaphoreType.DMA,
            pltpu.SemaphoreType.DMA,
            pltpu.SemaphoreType.DMA,
        ],
    )
    def body(enc_hbm, a0_hbm, a1_hbm, a2_hbm, tab_hbm, c0_hbm, c1_hbm,
             c2_hbm, w_hbm, out_hbm, idx_v, dmx_a, dmx_b, rows_a, rows_b,
             cat_idx, cat0_v, cat1_v, cat2_v, out_v, w_v, scores_v,
             sem_a, sem_b, sem_c):
        cid = lax.axis_index("c")
        sid = lax.axis_index("s")
        wid = sid * _NC + cid
        base = wid * BPW

        pltpu.sync_copy(enc_hbm.at[pl.ds(base * LP, BPW * LP)], idx_v)
        pltpu.sync_copy(w_hbm, w_v)

        pltpu.sync_copy(a0_hbm.at[pl.ds(base, BPW)], cat_idx)
        pltpu.async_copy(c0_hbm.at[cat_idx], cat0_v, sem_c).wait()
        pltpu.sync_copy(a1_hbm.at[pl.ds(base, BPW)], cat_idx)
        pltpu.async_copy(c1_hbm.at[cat_idx], cat1_v, sem_c).wait()
        pltpu.sync_copy(a2_hbm.at[pl.ds(base, BPW)], cat_idx)
        pltpu.async_copy(c2_hbm.at[cat_idx], cat2_v, sem_c).wait()

        w0 = w_v[pl.ds(0, _LANES)]
        w1 = w_v[pl.ds(16, _LANES)]
        w2 = w_v[pl.ds(32, _LANES)]
        w3 = w_v[pl.ds(48, _LANES)]
        lane = lax.iota(jnp.int32, _LANES)

        def fill_dmx(r, dmx):
            ib = r * LP
            def fb(i, _):
                iv = idx_v[pl.ds(ib + i * _LANES, _LANES)]
                a = iv + iv
                b = a - (2 * OFF - 1)
                dmx[pl.ds(i * _LANES, _LANES)] = jnp.where(iv < OFF, a, b)
                return 0
            lax.fori_loop(0, NV, fb, 0, unroll=False)

        def start_gather(buf, dmx, sem):
            pltpu.async_copy(tab_hbm.at[dmx.at[pl.ds(0, CH0)]],
                             buf.at[pl.ds(0, CH0)], sem)
            pltpu.async_copy(tab_hbm.at[dmx.at[pl.ds(CH0, CH1)]],
                             buf.at[pl.ds(CH0, CH1)], sem)

        def wait_gather(buf, dmx, sem):
            pltpu.make_async_copy(tab_hbm.at[dmx.at[pl.ds(0, CH0)]],
                                  buf.at[pl.ds(0, CH0)], sem).wait()
            pltpu.make_async_copy(tab_hbm.at[dmx.at[pl.ds(0, CH1)]],
                                  buf.at[pl.ds(CH0, CH1)], sem).wait()

        def rotate(v, sh):
            return jnp.take_along_axis(v, (lane + sh) & (_LANES - 1), axis=0)

        def allreduce_sum(v):
            for sh in (8, 4, 2, 1):
                v = v + rotate(v, sh)
            return v

        def allreduce_max(v):
            for sh in (8, 4, 2, 1):
                v = jnp.maximum(v, rotate(v, sh))
            return v

        def compute_row(buf, r):
            def score_group(i, n_tok):
                sv = jnp.full((_LANES,), _NEG, jnp.float32)
                for u in range(n_tok):
                    j = i * _LANES + u
                    v = (buf[j, pl.ds(0, 16)] * w0
                         + buf[j, pl.ds(16, 16)] * w1
                         + buf[j, pl.ds(32, 16)] * w2
                         + buf[j, pl.ds(48, 16)] * w3)
                    sv = jnp.where(lane == u, allreduce_sum(v), sv)
                scores_v[pl.ds(i * _LANES, _LANES)] = sv
                return sv

            def score_body(i, mv):
                return jnp.maximum(mv, score_group(i, _LANES))
            mv = lax.fori_loop(0, NFULL, score_body,
                               jnp.full((_LANES,), _NEG, jnp.float32),
                               unroll=False)
            mv = jnp.maximum(mv, score_group(NFULL, TAIL))
            m = allreduce_max(mv)

            def exp_body(i, sv):
                e = jnp.exp(scores_v[pl.ds(i * _LANES, _LANES)] - m)
                scores_v[pl.ds(i * _LANES, _LANES)] = e
                return sv + e
            sv = lax.fori_loop(0, NV, exp_body,
                               jnp.zeros((_LANES,), jnp.float32),
                               unroll=False)
            inv = 1.0 / allreduce_sum(sv)

            def wsum_group(i, n_tok, accs):
                e16 = scores_v[pl.ds(i * _LANES, _LANES)]
                a0v, a1v, a2v, a3v = accs
                for u in range(n_tok):
                    j = i * _LANES + u
                    wb = jnp.take_along_axis(
                        e16, jnp.full((_LANES,), u, jnp.int32), axis=0)
                    a0v = a0v + wb * buf[j, pl.ds(0, 16)]
                    a1v = a1v + wb * buf[j, pl.ds(16, 16)]
                    a2v = a2v + wb * buf[j, pl.ds(32, 16)]
                    a3v = a3v + wb * buf[j, pl.ds(48, 16)]
                return (a0v, a1v, a2v, a3v)
            z16 = jnp.zeros((_LANES,), jnp.float32)
            accs = lax.fori_loop(0, NFULL,
                                 lambda i, accs: wsum_group(i, _LANES, accs),
                                 (z16, z16, z16, z16), unroll=False)
            accs = wsum_group(NFULL, TAIL, accs)

            for g in range(DG):
                out_v[r, pl.ds(g * 16, 16)] = (
                    accs[g] * inv
                    + cat0_v[r, pl.ds(g * 16, 16)]
                    + cat1_v[r, pl.ds(g * 16, 16)]
                    + cat2_v[r, pl.ds(g * 16, 16)])

        fill_dmx(0, dmx_a)
        start_gather(rows_a, dmx_a, sem_a)
        fill_dmx(1, dmx_b)
        start_gather(rows_b, dmx_b, sem_b)

        def row_pair(k, _):
            r = k * 2
            wait_gather(rows_a, dmx_a, sem_a)
            compute_row(rows_a, r)

            @pl.when(r + 2 < BPW)
            def _():
                fill_dmx(r + 2, dmx_a)
                start_gather(rows_a, dmx_a, sem_a)

            wait_gather(rows_b, dmx_b, sem_b)
            compute_row(rows_b, r + 1)

            @pl.when(r + 3 < BPW)
            def _():
                fill_dmx(r + 3, dmx_b)
                start_gather(rows_b, dmx_b, sem_b)
            return 0

        lax.fori_loop(0, BPW // 2, row_pair, 0, unroll=False)

        pltpu.sync_copy(out_v, out_hbm.at[pl.ds(base, BPW)])

    return body(enc_pad, a0, a1, a2, tab64, cat0, cat1, cat2, w_flat)


def _tc_fc(x_in, fc_w, fc_b2):
    B, D = x_in.shape
    C = fc_w.shape[1]
    BLK = 512

    def body(x_ref, w_ref, b_ref, o_ref):
        o_ref[...] = (
            jnp.dot(x_ref[...], w_ref[...], preferred_element_type=jnp.float32)
            + b_ref[...])

    return pl.pallas_call(
        body,
        grid=(B // BLK,),
        in_specs=[
            pl.BlockSpec((BLK, D), lambda i: (i, 0)),
            pl.BlockSpec((D, C), lambda i: (0, 0)),
            pl.BlockSpec((1, C), lambda i: (0, 0)),
        ],
        out_specs=pl.BlockSpec((BLK, C), lambda i: (i, 0)),
        out_shape=jax.ShapeDtypeStruct((B, C), jnp.float32),
    )(x_in, fc_w, fc_b2)


def kernel(encoded_text, additional_inputs, emb_table, cat_table0, cat_table1,
           cat_table2, attn_w, fc_w, fc_b):
    B, L = encoded_text.shape
    V, D = emb_table.shape
    LP = ((L + 15) // 16) * 16
    enc = encoded_text.astype(jnp.int32)
    enc_pad = jnp.pad(enc, ((0, 0), (0, LP - L))).reshape(-1)
    a0 = additional_inputs[0].astype(jnp.int32)
    a1 = additional_inputs[1].astype(jnp.int32)
    a2 = additional_inputs[2].astype(jnp.int32)
    tab64, off = _repack_table(emb_table)
    w_flat = attn_w.reshape(-1)
    x_in = _sc_pool(enc_pad, a0, a1, a2, tab64, cat_table0, cat_table1,
                    cat_table2, w_flat, B, L, LP, D, off)
    return _tc_fc(x_in, fc_w, fc_b.reshape(1, -1))
